# baseline (speedup 1.0000x reference)
import functools

import jax
import jax.numpy as jnp
from jax import lax
from jax.experimental import pallas as pl
from jax.experimental.pallas import tpu as pltpu
from jax.experimental.pallas import tpu_sc as plsc

DIM = 32
SLICE = 128
SPC = 4
CHUNK = SLICE * SPC
NW = 32
NC = 2


def _rsqrt_vec(v):
    i = lax.bitcast_convert_type(v, jnp.int32)
    i = jnp.int32(0x5F3759DF) - lax.shift_right_logical(i, 1)
    y = lax.bitcast_convert_type(i, jnp.float32)
    y = y * (1.5 - 0.5 * v * y * y)
    y = y * (1.5 - 0.5 * v * y * y)
    y = y * (1.5 - 0.5 * v * y * y)
    return y


_GATHER_DNUMS = lax.GatherDimensionNumbers(
    offset_dims=(), collapsed_slice_dims=(0,), start_index_map=(0,))


def _lane_perm(v, p):
    return lax.gather(v, p[:, None], _GATHER_DNUMS, (1,),
                      mode=lax.GatherScatterMode.PROMISE_IN_BOUNDS)


def _lane_sum(v, perms):
    for p in perms:
        v = v + _lane_perm(v, p)
    return v


def _make_sc_kernel(n_rows):
    assert n_rows % (NW * CHUNK) == 0
    chunks_per_w = n_rows // (NW * CHUNK)
    mesh = plsc.VectorSubcoreMesh(core_axis_name="c", subcore_axis_name="s")

    @functools.partial(
        pl.kernel,
        mesh=mesh,
        out_type=jax.ShapeDtypeStruct((n_rows, DIM), jnp.float32),
        scratch_types=[
            pltpu.VMEM((SPC, SLICE), jnp.int32),
            pltpu.VMEM((SPC, SLICE), jnp.int32),
            pltpu.VMEM((CHUNK, DIM), jnp.float32),
            pltpu.VMEM((CHUNK, DIM), jnp.float32),
            pltpu.VMEM((DIM,), jnp.float32),
            pltpu.VMEM((DIM,), jnp.float32),
            pltpu.SemaphoreType.DMA,
            pltpu.SemaphoreType.DMA,
        ],
        compiler_params=pltpu.CompilerParams(use_tc_tiling_on_sc=False),
    )
    def body(x2d, table, gamma, beta, out,
             idx0, idx1, rows0, rows1, g_v, b_v, sem0, sem1):
        c = lax.axis_index("c")
        s = lax.axis_index("s")
        wid = s * NC + c
        pltpu.sync_copy(gamma, g_v)
        pltpu.sync_copy(beta, b_v)
        g0 = g_v[pl.ds(0, 16)]
        g1 = g_v[pl.ds(16, 16)]
        b0 = b_v[pl.ds(0, 16)]
        b1 = b_v[pl.ds(16, 16)]
        lanes = lax.iota(jnp.int32, 16)
        perms = [jnp.bitwise_xor(lanes, jnp.int32(k)) for k in (1, 2, 4, 8)]
        base = wid * chunks_per_w
        bufs = ((idx0, rows0, sem0), (idx1, rows1, sem1))

        def stage(ch, idx_v, rows_v, sem):
            pltpu.sync_copy(x2d.at[pl.ds(ch * SPC, SPC)], idx_v)
            for j in range(SPC):
                pltpu.async_copy(table.at[idx_v.at[j]],
                                 rows_v.at[pl.ds(j * SLICE, SLICE)], sem)

        def drain(idx_v, rows_v, sem):
            for j in range(SPC):
                pltpu.make_async_copy(table.at[idx_v.at[j]],
                                      rows_v.at[pl.ds(j * SLICE, SLICE)],
                                      sem).wait()

        def compute_out(ch, rows_v):
            def row_body(r, carry):
                v0 = rows_v[r, pl.ds(0, 16)]
                v1 = rows_v[r, pl.ds(16, 16)]
                sm = _lane_sum(v0 + v1, perms)
                sq = _lane_sum(v0 * v0 + v1 * v1, perms)
                mu = sm * (1.0 / DIM)
                var = sq * (1.0 / DIM) - mu * mu
                var = jnp.maximum(var, 0.0) + 1e-5
                rs = _rsqrt_vec(var)
                rows_v[r, pl.ds(0, 16)] = (v0 - mu) * (rs * g0) + b0
                rows_v[r, pl.ds(16, 16)] = (v1 - mu) * (rs * g1) + b1
                return carry

            lax.fori_loop(0, CHUNK, row_body, 0, unroll=8)
            pltpu.sync_copy(rows_v, out.at[pl.ds(ch * CHUNK, CHUNK)])

        def step(ch, buf, fire_next):
            idx_v, rows_v, sem = bufs[buf]
            nidx, nrows, nsem = bufs[1 - buf]
            if fire_next is not None:
                stage(fire_next, nidx, nrows, nsem)
            drain(idx_v, rows_v, sem)
            compute_out(ch, rows_v)

        stage(base, idx0, rows0, sem0)

        def loop_t(t, carry):
            ch = base + 2 * t
            step(ch, 0, ch + 1)
            step(ch + 1, 1, ch + 2)
            return carry

        if chunks_per_w > 2:
            lax.fori_loop(0, chunks_per_w // 2 - 1, loop_t, 0)
        last = base + chunks_per_w - 2
        step(last, 0, last + 1)
        step(last + 1, 1, None)

    return body


def kernel(x, table, gamma, beta):
    b, h = x.shape
    n_rows = b * h
    x2d = x.reshape(-1).astype(jnp.int32).reshape(n_rows // SLICE, SLICE)
    out = _make_sc_kernel(n_rows)(x2d, table, gamma, beta)
    return out.reshape(b, h, DIM)


# speedup vs baseline: 2.4244x; 2.4244x over previous
import functools

import jax
import jax.numpy as jnp
from jax import lax
from jax.experimental import pallas as pl
from jax.experimental.pallas import tpu as pltpu
from jax.experimental.pallas import tpu_sc as plsc

DIM = 32
SLICE = 128
SPC = 4
CHUNK = SLICE * SPC
NW = 32
NC = 2


def _rsqrt_vec(v):
    i = lax.bitcast_convert_type(v, jnp.int32)
    i = jnp.int32(0x5F3759DF) - lax.shift_right_logical(i, 1)
    y = lax.bitcast_convert_type(i, jnp.float32)
    y = y * (1.5 - 0.5 * v * y * y)
    y = y * (1.5 - 0.5 * v * y * y)
    y = y * (1.5 - 0.5 * v * y * y)
    return y


_GATHER_DNUMS = lax.GatherDimensionNumbers(
    offset_dims=(), collapsed_slice_dims=(0,), start_index_map=(0,))


def _lane_perm(v, p):
    return lax.gather(v, p[:, None], _GATHER_DNUMS, (1,),
                      mode=lax.GatherScatterMode.PROMISE_IN_BOUNDS)


def _lane_sum(v, perms):
    for p in perms:
        v = v + _lane_perm(v, p)
    return v


def _make_sc_kernel(batch, hist):
    n_tasks = hist * (batch // SLICE)
    assert n_tasks % NW == 0 and (n_tasks // NW) % SPC == 0
    tasks_per_w = n_tasks // NW
    chunks_per_w = tasks_per_w // SPC
    mesh = plsc.VectorSubcoreMesh(core_axis_name="c", subcore_axis_name="s")

    @functools.partial(
        pl.kernel,
        mesh=mesh,
        out_type=jax.ShapeDtypeStruct((batch, hist * DIM), jnp.float32),
        scratch_types=[
            pltpu.VMEM((SPC, SLICE), jnp.int32),
            pltpu.VMEM((SPC, SLICE), jnp.int32),
            pltpu.VMEM((CHUNK, DIM), jnp.float32),
            pltpu.VMEM((CHUNK, DIM), jnp.float32),
            pltpu.VMEM((DIM,), jnp.float32),
            pltpu.VMEM((DIM,), jnp.float32),
            pltpu.SemaphoreType.DMA,
            pltpu.SemaphoreType.DMA,
        ],
        compiler_params=pltpu.CompilerParams(use_tc_tiling_on_sc=False),
    )
    def body(xt, table, gamma, beta, out,
             idx0, idx1, rows0, rows1, g_v, b_v, sem0, sem1):
        c = lax.axis_index("c")
        s = lax.axis_index("s")
        wid = s * NC + c
        pltpu.sync_copy(gamma, g_v)
        pltpu.sync_copy(beta, b_v)
        g0 = g_v[pl.ds(0, 16)]
        g1 = g_v[pl.ds(16, 16)]
        b0 = b_v[pl.ds(0, 16)]
        b1 = b_v[pl.ds(16, 16)]
        lanes = lax.iota(jnp.int32, 16)
        perms = [jnp.bitwise_xor(lanes, jnp.int32(k)) for k in (1, 2, 4, 8)]
        base = wid * tasks_per_w
        bufs = ((idx0, rows0, sem0), (idx1, rows1, sem1))

        def stage(t0, idx_v, rows_v, sem):
            for j in range(SPC):
                t = t0 + j
                h = lax.shift_right_logical(t, 7)
                bblk = lax.bitwise_and(t, SLICE - 1)
                pltpu.sync_copy(xt.at[h, pl.ds(bblk * SLICE, SLICE)],
                                idx_v.at[j])
            for j in range(SPC):
                pltpu.async_copy(table.at[idx_v.at[j]],
                                 rows_v.at[pl.ds(j * SLICE, SLICE)], sem)

        def drain(idx_v, rows_v, sem):
            for j in range(SPC):
                pltpu.make_async_copy(table.at[idx_v.at[j]],
                                      rows_v.at[pl.ds(j * SLICE, SLICE)],
                                      sem).wait()

        def compute_out(t0, rows_v):
            @plsc.parallel_loop(0, CHUNK, step=1, unroll=8)
            def _row(r):
                v0 = rows_v[r, pl.ds(0, 16)]
                v1 = rows_v[r, pl.ds(16, 16)]
                sm = _lane_sum(v0 + v1, perms)
                sq = _lane_sum(v0 * v0 + v1 * v1, perms)
                mu = sm * (1.0 / DIM)
                var = sq * (1.0 / DIM) - mu * mu
                var = jnp.maximum(var, 0.0) + 1e-5
                rs = _rsqrt_vec(var)
                rows_v[r, pl.ds(0, 16)] = (v0 - mu) * (rs * g0) + b0
                rows_v[r, pl.ds(16, 16)] = (v1 - mu) * (rs * g1) + b1

            for j in range(SPC):
                t = t0 + j
                h = lax.shift_right_logical(t, 7)
                bblk = lax.bitwise_and(t, SLICE - 1)
                pltpu.sync_copy(
                    rows_v.at[pl.ds(j * SLICE, SLICE)],
                    out.at[pl.ds(bblk * SLICE, SLICE),
                           pl.ds(h * DIM, DIM)])

        def step(t0, buf, fire_next):
            idx_v, rows_v, sem = bufs[buf]
            nidx, nrows, nsem = bufs[1 - buf]
            if fire_next is not None:
                stage(fire_next, nidx, nrows, nsem)
            drain(idx_v, rows_v, sem)
            compute_out(t0, rows_v)

        stage(base, idx0, rows0, sem0)

        def loop_t(i, carry):
            t0 = base + 2 * SPC * i
            step(t0, 0, t0 + SPC)
            step(t0 + SPC, 1, t0 + 2 * SPC)
            return carry

        if chunks_per_w > 2:
            lax.fori_loop(0, chunks_per_w // 2 - 1, loop_t, 0)
        last = base + (chunks_per_w - 2) * SPC
        step(last, 0, last + SPC)
        step(last + SPC, 1, None)

    return body


def kernel(x, table, gamma, beta):
    b, h = x.shape
    xt = x.astype(jnp.int32).T
    out = _make_sc_kernel(b, h)(xt, table, gamma, beta)
    return out.reshape(b, h, DIM)
